# baseline (speedup 1.0000x reference)
import jax
import jax.numpy as jnp
from jax.experimental import pallas as pl
from jax.experimental.pallas import tpu as pltpu

D_MODEL = 768
D_STATE = 16
D_CONV = 4
D_INNER = 1536
DT_RANK = 48
SEQ = 2048
BLK_L = 512
N_BLK = SEQ // BLK_L
UNROLL = 16


def _silu(x):
    return x * jax.nn.sigmoid(x)


def _front_fwd_kernel(x_ref, w_in_ref, conv_w_ref, conv_b_ref, w_x_ref,
                      w_dt_ref, b_dt_ref, at_ref,
                      h_ref, g_ref, dt_ref, bc_ref, scof_ref,
                      carry_ref, xst_ref, u_ref):
    i = pl.program_id(0)
    x = x_ref[...]
    proj = jnp.dot(x, w_in_ref[...], preferred_element_type=jnp.float32)
    hidden = proj[:, :D_INNER]
    gate = proj[:, D_INNER:]

    @pl.when(i == 0)
    def _():
        carry_ref[...] = jnp.zeros_like(carry_ref)
        xst_ref[...] = jnp.zeros_like(xst_ref)

    hp = jnp.concatenate([carry_ref[...], hidden], axis=0)
    conv = jnp.broadcast_to(conv_b_ref[...], (BLK_L, D_INNER))
    for k in range(D_CONV):
        conv = conv + conv_w_ref[k:k + 1, :] * hp[k:k + BLK_L, :]
    carry_ref[...] = hidden[BLK_L - (D_CONV - 1):, :]

    h = _silu(conv)
    ssm_p = jnp.dot(h, w_x_ref[...], preferred_element_type=jnp.float32)
    ts = ssm_p[:, :DT_RANK]
    dt = jax.nn.softplus(
        jnp.dot(ts, w_dt_ref[...], preferred_element_type=jnp.float32)
        + b_dt_ref[...])
    h_ref[...] = h
    g_ref[...] = _silu(gate)
    dt_ref[...] = dt
    u_ref[...] = dt * h
    bc_ref[...] = ssm_p[:, DT_RANK:]

    at = at_ref[...]

    def body(r, xf):
        dtrow = dt_ref[pl.ds(r, 1), :]
        urow = u_ref[pl.ds(r, 1), :]
        bccol = jnp.transpose(bc_ref[pl.ds(r, 1), :])
        bcol = bccol[:D_STATE, :]
        ccol = bccol[D_STATE:, :]
        xf = jnp.exp(at * dtrow) * xf + bcol * urow
        scof_ref[pl.ds(r, 1), :] = jnp.sum(xf * ccol, axis=0, keepdims=True)
        return xf

    xst_ref[...] = jax.lax.fori_loop(0, BLK_L, body, xst_ref[...],
                                     unroll=UNROLL)


def _bwd_out_kernel(dt_ref, bc_ref, scof_ref, h_ref, g_ref,
                    at_ref, d_ref, w_out_ref, out_ref,
                    xst_ref, scob_ref, u_ref):
    i = pl.program_id(0)

    @pl.when(i == 0)
    def _():
        xst_ref[...] = jnp.zeros_like(xst_ref)

    u_ref[...] = dt_ref[...] * h_ref[...]
    at = at_ref[...]

    def body(j, s):
        r = BLK_L - 1 - j
        dtrow = dt_ref[pl.ds(r, 1), :]
        urow = u_ref[pl.ds(r, 1), :]
        bccol = jnp.transpose(bc_ref[pl.ds(r, 1), :])
        bcol = bccol[:D_STATE, :]
        ccol = bccol[D_STATE:, :]
        xb = bcol * urow + s
        scob_ref[pl.ds(r, 1), :] = jnp.sum(xb * ccol, axis=0, keepdims=True)
        return jnp.exp(at * dtrow) * xb

    xst_ref[...] = jax.lax.fori_loop(0, BLK_L, body, xst_ref[...],
                                     unroll=UNROLL)

    bc = bc_ref[...]
    cb = jnp.sum(bc[:, :D_STATE] * bc[:, D_STATE:], axis=1, keepdims=True)
    y = (1.3 * (scof_ref[...] + scob_ref[...] - cb * u_ref[...])
         + h_ref[...] * d_ref[...]) * g_ref[...]
    out_ref[...] = jnp.dot(y, w_out_ref[...], preferred_element_type=jnp.float32)


def kernel(input_states, context_len, W_in, conv_w, conv_b, W_x, W_dt, b_dt,
           A_log, D, W_out):
    del context_len
    x = input_states[0]
    conv_w_t = conv_w.T
    at = -jnp.exp(A_log).T

    full = lambda shape: pl.BlockSpec(shape, lambda i: (0, 0))
    fwd_blk = lambda w: pl.BlockSpec((BLK_L, w), lambda i: (i, 0))
    bwd_blk = lambda w: pl.BlockSpec((BLK_L, w), lambda i: (N_BLK - 1 - i, 0))
    f32 = jnp.float32

    h, g, dt, bc, scof = pl.pallas_call(
        _front_fwd_kernel,
        grid=(N_BLK,),
        in_specs=[
            fwd_blk(D_MODEL),
            full((D_MODEL, 2 * D_INNER)),
            full((D_CONV, D_INNER)),
            full((1, D_INNER)),
            full((D_INNER, DT_RANK + 2 * D_STATE)),
            full((DT_RANK, D_INNER)),
            full((1, D_INNER)),
            full((D_STATE, D_INNER)),
        ],
        out_specs=[fwd_blk(D_INNER)] * 3
        + [fwd_blk(2 * D_STATE), fwd_blk(D_INNER)],
        out_shape=[jax.ShapeDtypeStruct((SEQ, D_INNER), f32)] * 3
        + [jax.ShapeDtypeStruct((SEQ, 2 * D_STATE), f32),
           jax.ShapeDtypeStruct((SEQ, D_INNER), f32)],
        scratch_shapes=[pltpu.VMEM((D_CONV - 1, D_INNER), f32),
                        pltpu.VMEM((D_STATE, D_INNER), f32),
                        pltpu.VMEM((BLK_L, D_INNER), f32)],
    )(x, W_in, conv_w_t, conv_b[None, :], W_x, W_dt, b_dt[None, :], at)

    out = pl.pallas_call(
        _bwd_out_kernel,
        grid=(N_BLK,),
        in_specs=[
            bwd_blk(D_INNER),
            bwd_blk(2 * D_STATE),
            bwd_blk(D_INNER),
            bwd_blk(D_INNER),
            bwd_blk(D_INNER),
            full((D_STATE, D_INNER)),
            full((1, D_INNER)),
            full((D_INNER, D_MODEL)),
        ],
        out_specs=bwd_blk(D_MODEL),
        out_shape=jax.ShapeDtypeStruct((SEQ, D_MODEL), f32),
        scratch_shapes=[pltpu.VMEM((D_STATE, D_INNER), f32),
                        pltpu.VMEM((BLK_L, D_INNER), f32),
                        pltpu.VMEM((BLK_L, D_INNER), f32)],
    )(dt, bc, scof, h, g, at, D[None, :], W_out)

    return out[None]


# speedup vs baseline: 1.0032x; 1.0032x over previous
import jax
import jax.numpy as jnp
from jax.experimental import pallas as pl
from jax.experimental.pallas import tpu as pltpu

D_MODEL = 768
D_STATE = 16
D_CONV = 4
D_INNER = 1536
DT_RANK = 48
SEQ = 2048
BLK_L = 256
N_BLK = SEQ // BLK_L
UNROLL = 16


def _silu(x):
    return x * jax.nn.sigmoid(x)


def _front_fwd_kernel(x_ref, w_in_ref, conv_w_ref, conv_b_ref, w_x_ref,
                      w_dt_ref, b_dt_ref, at_ref,
                      h_ref, g_ref, dt_ref, bc_ref, scof_ref,
                      carry_ref, xst_ref, u_ref):
    i = pl.program_id(0)
    x = x_ref[...]
    proj = jnp.dot(x, w_in_ref[...], preferred_element_type=jnp.float32)
    hidden = proj[:, :D_INNER]
    gate = proj[:, D_INNER:]

    @pl.when(i == 0)
    def _():
        carry_ref[...] = jnp.zeros_like(carry_ref)
        xst_ref[...] = jnp.zeros_like(xst_ref)

    hp = jnp.concatenate([carry_ref[...], hidden], axis=0)
    conv = jnp.broadcast_to(conv_b_ref[...], (BLK_L, D_INNER))
    for k in range(D_CONV):
        conv = conv + conv_w_ref[k:k + 1, :] * hp[k:k + BLK_L, :]
    carry_ref[...] = hidden[BLK_L - (D_CONV - 1):, :]

    h = _silu(conv)
    ssm_p = jnp.dot(h, w_x_ref[...], preferred_element_type=jnp.float32)
    ts = ssm_p[:, :DT_RANK]
    dt = jax.nn.softplus(
        jnp.dot(ts, w_dt_ref[...], preferred_element_type=jnp.float32)
        + b_dt_ref[...])
    h_ref[...] = h
    g_ref[...] = _silu(gate)
    dt_ref[...] = dt
    u_ref[...] = dt * h
    bc_ref[...] = ssm_p[:, DT_RANK:]

    at = at_ref[...]

    def body(r, xf):
        dtrow = dt_ref[pl.ds(r, 1), :]
        urow = u_ref[pl.ds(r, 1), :]
        bccol = jnp.transpose(bc_ref[pl.ds(r, 1), :])
        bcol = bccol[:D_STATE, :]
        ccol = bccol[D_STATE:, :]
        xf = jnp.exp(at * dtrow) * xf + bcol * urow
        scof_ref[pl.ds(r, 1), :] = jnp.sum(xf * ccol, axis=0, keepdims=True)
        return xf

    xst_ref[...] = jax.lax.fori_loop(0, BLK_L, body, xst_ref[...],
                                     unroll=UNROLL)


def _bwd_out_kernel(dt_ref, bc_ref, scof_ref, h_ref, g_ref,
                    at_ref, d_ref, w_out_ref, out_ref,
                    xst_ref, scob_ref, u_ref):
    i = pl.program_id(0)

    @pl.when(i == 0)
    def _():
        xst_ref[...] = jnp.zeros_like(xst_ref)

    u_ref[...] = dt_ref[...] * h_ref[...]
    at = at_ref[...]

    def body(j, s):
        r = BLK_L - 1 - j
        dtrow = dt_ref[pl.ds(r, 1), :]
        urow = u_ref[pl.ds(r, 1), :]
        bccol = jnp.transpose(bc_ref[pl.ds(r, 1), :])
        bcol = bccol[:D_STATE, :]
        ccol = bccol[D_STATE:, :]
        xb = bcol * urow + s
        scob_ref[pl.ds(r, 1), :] = jnp.sum(xb * ccol, axis=0, keepdims=True)
        return jnp.exp(at * dtrow) * xb

    xst_ref[...] = jax.lax.fori_loop(0, BLK_L, body, xst_ref[...],
                                     unroll=UNROLL)

    bc = bc_ref[...]
    cb = jnp.sum(bc[:, :D_STATE] * bc[:, D_STATE:], axis=1, keepdims=True)
    y = (1.3 * (scof_ref[...] + scob_ref[...] - cb * u_ref[...])
         + h_ref[...] * d_ref[...]) * g_ref[...]
    out_ref[...] = jnp.dot(y, w_out_ref[...], preferred_element_type=jnp.float32)


def kernel(input_states, context_len, W_in, conv_w, conv_b, W_x, W_dt, b_dt,
           A_log, D, W_out):
    del context_len
    x = input_states[0]
    conv_w_t = conv_w.T
    at = -jnp.exp(A_log).T

    full = lambda shape: pl.BlockSpec(shape, lambda i: (0, 0))
    fwd_blk = lambda w: pl.BlockSpec((BLK_L, w), lambda i: (i, 0))
    bwd_blk = lambda w: pl.BlockSpec((BLK_L, w), lambda i: (N_BLK - 1 - i, 0))
    f32 = jnp.float32

    h, g, dt, bc, scof = pl.pallas_call(
        _front_fwd_kernel,
        grid=(N_BLK,),
        in_specs=[
            fwd_blk(D_MODEL),
            full((D_MODEL, 2 * D_INNER)),
            full((D_CONV, D_INNER)),
            full((1, D_INNER)),
            full((D_INNER, DT_RANK + 2 * D_STATE)),
            full((DT_RANK, D_INNER)),
            full((1, D_INNER)),
            full((D_STATE, D_INNER)),
        ],
        out_specs=[fwd_blk(D_INNER)] * 3
        + [fwd_blk(2 * D_STATE), fwd_blk(D_INNER)],
        out_shape=[jax.ShapeDtypeStruct((SEQ, D_INNER), f32)] * 3
        + [jax.ShapeDtypeStruct((SEQ, 2 * D_STATE), f32),
           jax.ShapeDtypeStruct((SEQ, D_INNER), f32)],
        scratch_shapes=[pltpu.VMEM((D_CONV - 1, D_INNER), f32),
                        pltpu.VMEM((D_STATE, D_INNER), f32),
                        pltpu.VMEM((BLK_L, D_INNER), f32)],
    )(x, W_in, conv_w_t, conv_b[None, :], W_x, W_dt, b_dt[None, :], at)

    out = pl.pallas_call(
        _bwd_out_kernel,
        grid=(N_BLK,),
        in_specs=[
            bwd_blk(D_INNER),
            bwd_blk(2 * D_STATE),
            bwd_blk(D_INNER),
            bwd_blk(D_INNER),
            bwd_blk(D_INNER),
            full((D_STATE, D_INNER)),
            full((1, D_INNER)),
            full((D_INNER, D_MODEL)),
        ],
        out_specs=bwd_blk(D_MODEL),
        out_shape=jax.ShapeDtypeStruct((SEQ, D_MODEL), f32),
        scratch_shapes=[pltpu.VMEM((D_STATE, D_INNER), f32),
                        pltpu.VMEM((BLK_L, D_INNER), f32),
                        pltpu.VMEM((BLK_L, D_INNER), f32)],
    )(dt, bc, scof, h, g, at, D[None, :], W_out)

    return out[None]
